# baseline (speedup 1.0000x reference)
import functools

import jax
import jax.numpy as jnp
from jax.experimental import pallas as pl
from jax.experimental.pallas import tpu as pltpu

NUM_TYPES = 10
N = 100000
R = 8


def _kernel(x_ref, act_ref, cum_ref, lp_ref, sel_ref, ent_ref):
    x = x_ref[...]
    m = jnp.max(x, axis=1, keepdims=True)
    s = x - m
    e = jnp.exp(s)
    z = jnp.sum(e, axis=1, keepdims=True)
    es = jnp.sum(e * s, axis=1, keepdims=True)
    lz = jnp.log(z)
    lp_ref[...] = s - lz
    ent_ref[...] = lz - es / z

    at = act_ref[:, 0:1]
    ap = act_ref[:, 1:2]
    cum = cum_ref[...]
    tix = jax.lax.broadcasted_iota(jnp.int32, (R, NUM_TYPES + 1), 1)
    cum_at = jnp.sum(jnp.where(tix == at, cum, 0), axis=1, keepdims=True)
    idx = cum_at + at + ap

    lanes = jax.lax.broadcasted_iota(jnp.int32, (R, N), 1)
    sel_s = jnp.sum(jnp.where(lanes == idx, s, 0.0), axis=1, keepdims=True)
    sel_ref[...] = sel_s - lz


@jax.jit
def kernel(logits, action, cum_action_max_params):
    b = logits.shape[0]
    cum2d = cum_action_max_params.reshape(1, NUM_TYPES + 1)
    grid = (b // R,)
    lp, sel, ent = pl.pallas_call(
        _kernel,
        grid=grid,
        in_specs=[
            pl.BlockSpec((R, N), lambda i: (i, 0)),
            pl.BlockSpec((R, 2), lambda i: (i, 0)),
            pl.BlockSpec((1, NUM_TYPES + 1), lambda i: (0, 0)),
        ],
        out_specs=[
            pl.BlockSpec((R, N), lambda i: (i, 0)),
            pl.BlockSpec((R, 1), lambda i: (i, 0)),
            pl.BlockSpec((R, 1), lambda i: (i, 0)),
        ],
        out_shape=[
            jax.ShapeDtypeStruct((b, N), jnp.float32),
            jax.ShapeDtypeStruct((b, 1), jnp.float32),
            jax.ShapeDtypeStruct((b, 1), jnp.float32),
        ],
        compiler_params=pltpu.CompilerParams(
            dimension_semantics=("arbitrary",),
        ),
    )(logits, action.astype(jnp.int32), cum2d)
    return sel[:, 0], ent[:, 0], lp
